# baseline (speedup 1.0000x reference)
import functools

import jax
import jax.numpy as jnp
from jax import lax
from jax.experimental import pallas as pl
from jax.experimental.pallas import tpu as pltpu
from jax.experimental.pallas import tpu_sc as plsc

_OFFSETS = (-2, -1, 1, 2)
_NSUB = 32
_LANES = 16


def _sc_band_body(nb, vs, nsub, nmax, ni_hbm, nw_hbm, *refs):
    nslots = len(_OFFSETS)
    w_hbm = refs[:nslots]
    ni_v, nw_v = refs[nslots], refs[nslots + 1]
    w_v = refs[nslots + 2 :]
    wid = lax.axis_index("s") * 2 + lax.axis_index("c")

    @pl.when(wid < nsub)
    def _strip():
        v0 = wid * vs
        pltpu.sync_copy(ni_hbm.at[pl.ds(v0, vs), :], ni_v)
        pltpu.sync_copy(nw_hbm.at[pl.ds(v0, vs), :], nw_v)

        lanes = lax.iota(jnp.int32, _LANES)

        def chunk(c, carry):
            base = c * _LANES
            vloc = base + lanes
            vabs = v0 + vloc
            accs = [jnp.zeros((_LANES,), jnp.float32) for _ in _OFFSETS]
            for k in range(nmax):
                col = jnp.full((_LANES,), k, jnp.int32)
                u = plsc.load_gather(ni_v, [vloc, col])
                w = plsc.load_gather(nw_v, [vloc, col])
                diff = u - vabs
                diff = jnp.where(diff > 2, diff - nb, diff)
                diff = jnp.where(diff < -2, diff + nb, diff)
                for slot, o in enumerate(_OFFSETS):
                    accs[slot] = accs[slot] + jnp.where(diff == o, w, 0.0)
            for slot in range(nslots):
                w_v[slot][pl.ds(base, _LANES)] = accs[slot]
            return carry

        lax.fori_loop(0, vs // _LANES, chunk, 0)
        for slot in range(nslots):
            pltpu.sync_copy(w_v[slot], w_hbm[slot].at[pl.ds(v0, vs)])


def _band_weights_sc(ni, nw):
    v, nmax = ni.shape
    nsub = _NSUB
    while v % (nsub * _LANES) != 0:
        nsub -= 1
    vs = v // nsub
    mesh = plsc.VectorSubcoreMesh(core_axis_name="c", subcore_axis_name="s")
    k = functools.partial(
        pl.kernel,
        mesh=mesh,
        out_type=[jax.ShapeDtypeStruct((v,), jnp.float32) for _ in _OFFSETS],
        scratch_types=[
            pltpu.VMEM((vs, nmax), jnp.int32),
            pltpu.VMEM((vs, nmax), jnp.float32),
        ]
        + [pltpu.VMEM((vs,), jnp.float32) for _ in _OFFSETS],
        compiler_params=pltpu.CompilerParams(needs_layout_passes=False),
    )(functools.partial(_sc_band_body, v, vs, nsub, nmax))
    return k(ni, nw)


def _roll_v(x, s):
    n = x.shape[-1]
    s = s % n
    if s == 0:
        return x
    return jnp.concatenate([x[..., s:], x[..., :s]], axis=-1)


def _loss_body(total, w0_ref, w1_ref, w2_ref, w3_ref, out_ref, tgt_ref, loss_ref):
    i = pl.program_id(0)

    @pl.when(i == 0)
    def _init():
        loss_ref[0, 0] = 0.0

    d = out_ref[...] - tgt_ref[...]
    acc = d
    for w_ref, o in zip((w0_ref, w1_ref, w2_ref, w3_ref), _OFFSETS):
        w = w_ref[...]
        acc = acc + w[None, None, :] * _roll_v(d, o)
    loss_ref[0, 0] += jnp.sum(acc * acc)

    @pl.when(i == pl.num_programs(0) - 1)
    def _final():
        loss_ref[0, 0] = loss_ref[0, 0] / total


def kernel(out, target, neighbor_idxs, neighbor_weights):
    b, nb, c = out.shape
    out3 = jnp.transpose(out, (2, 0, 1))
    tgt3 = jnp.transpose(target, (2, 0, 1))
    ws = _band_weights_sc(neighbor_idxs, neighbor_weights)

    bb = 8
    grid = (b // bb,)
    total = float(b * nb * c)
    res = pl.pallas_call(
        functools.partial(_loss_body, total),
        grid=grid,
        in_specs=[pl.BlockSpec((nb,), lambda i: (0,)) for _ in _OFFSETS]
        + [
            pl.BlockSpec((c, bb, nb), lambda i: (0, i, 0)),
            pl.BlockSpec((c, bb, nb), lambda i: (0, i, 0)),
        ],
        out_specs=pl.BlockSpec(
            (1, 1), lambda i: (0, 0), memory_space=pltpu.SMEM
        ),
        out_shape=jax.ShapeDtypeStruct((1, 1), jnp.float32),
    )(*ws, out3, tgt3)
    return jnp.reshape(res, ())
